# baseline (speedup 1.0000x reference)
import jax
import jax.numpy as jnp
from jax import lax
from jax.experimental import pallas as pl
from jax.experimental.pallas import tpu as pltpu
from jax.experimental.pallas import tpu_sc as plsc

B, D, K, C = 2048, 64, 512, 100
GAMMA = 0.99
EPS = 1e-05

NS = 16
L = 16
BPW = B // NS
CP = 128
HP = K * CP
HSL = HP // NS
DP = 128


def _tc1_body(x_ref, w_ref, am_ref, wtp_ref):
    x = x_ref[...]
    w = w_ref[...]

    xw = lax.dot_general(x, w, (((1,), (0,)), ((), ())),
                         precision=lax.Precision.HIGHEST)
    x2 = jnp.sum(x * x, axis=1, keepdims=True)
    w2 = jnp.sum(w * w, axis=0, keepdims=True)
    d2 = jnp.maximum(x2 - 2.0 * xw + w2, 0.0)
    dist = jnp.sqrt(d2)
    am = jnp.argmin(dist, axis=1)
    am_ref[...] = am

    wtp_ref[:, :D] = lax.transpose(w, (1, 0))


def _tc2_body(x_ref, am_ref, cs_ref, ea_ref, hist_ref, bh_ref, resp_ref,
              wn_ref, csn_ref, ean_ref, hn_ref, res_ref):
    x = x_ref[...]
    am_col = am_ref[...][:, None]
    onehot = (am_col == lax.broadcasted_iota(jnp.int32, (1, K), 1)
              ).astype(jnp.float32)

    n_idx = jnp.sum(onehot, axis=0, keepdims=True)
    n_idx = jnp.where(n_idx == 0.0, 1.0, n_idx)
    cs_new = cs_ref[...][None, :] * GAMMA + (1.0 - GAMMA) * n_idx
    csn_ref[...] = cs_new[0]

    embed_sum = lax.dot_general(
        x, onehot, (((0,), (0,)), ((), ())),
        precision=lax.Precision.HIGHEST)
    ea_new = ea_ref[...] * GAMMA + (1.0 - GAMMA) * embed_sum
    ean_ref[...] = ea_new

    n = jnp.sum(cs_new)
    cs_smoothed = (cs_new + EPS) / (n + K * EPS) * n
    wn_ref[...] = ea_new / cs_smoothed

    hn_ref[...] = hist_ref[...] * GAMMA + (1.0 - GAMMA) * bh_ref[:, :C]

    res_ref[...] = resp_ref[:, :D]


def _sc_body(am_hbm, keys_hbm, wt_hbm,
             resp_hbm, bh_hbm,
             idx_v, flat_v, ones_v, rows_v, a_v, acc_sh, sem):
    cid = lax.axis_index("c")
    sid = lax.axis_index("s")
    base = sid * BPW

    @pl.when(cid == 1)
    def _gather():
        pltpu.sync_copy(am_hbm.at[pl.ds(base, BPW)], idx_v)
        pltpu.async_copy(wt_hbm.at[idx_v], rows_v, sem).wait()
        pltpu.sync_copy(rows_v, resp_hbm.at[pl.ds(base, BPW)])

    @pl.when(cid == 0)
    def _hist():
        pltpu.sync_copy(am_hbm.at[pl.ds(base, BPW)], idx_v)
        pltpu.sync_copy(keys_hbm.at[pl.ds(base, BPW)], flat_v)
        for j in range(BPW // L):
            s = pl.ds(j * L, L)
            flat_v[s] = idx_v[s] * CP + flat_v[s]
            ones_v[s] = jnp.full((L,), 1.0, jnp.float32)

        def _zero(i, _):
            a_v[pl.ds(i * L, L)] = jnp.zeros((L,), jnp.float32)
            return 0
        lax.fori_loop(0, HSL // L, _zero, 0)
        esl = pl.ds(sid * HSL, HSL)
        pltpu.sync_copy(a_v, acc_sh.at[esl])
        plsc.subcore_barrier()
        pltpu.sync_copy(ones_v, acc_sh.at[flat_v], add=True)
        plsc.subcore_barrier()
        pltpu.sync_copy(acc_sh.at[esl], a_v)
        pltpu.sync_copy(a_v, bh_hbm.at[esl])


_sc_call = pl.kernel(
    _sc_body,
    out_type=(jax.ShapeDtypeStruct((B, DP), jnp.float32),
              jax.ShapeDtypeStruct((HP,), jnp.float32)),
    mesh=plsc.VectorSubcoreMesh(core_axis_name="c", subcore_axis_name="s"),
    scratch_types=[
        pltpu.VMEM((BPW,), jnp.int32),
        pltpu.VMEM((BPW,), jnp.int32),
        pltpu.VMEM((BPW,), jnp.float32),
        pltpu.VMEM((BPW, DP), jnp.float32),
        pltpu.VMEM((HSL,), jnp.float32),
        pltpu.VMEM_SHARED((HP,), jnp.float32),
        pltpu.SemaphoreType.DMA,
    ],
)


def kernel(batch_vectors, batch_keys_id, weight, cluster_size, embed_avg, hist):
    am, wt = pl.pallas_call(
        _tc1_body,
        out_shape=(
            jax.ShapeDtypeStruct((B,), jnp.int32),
            jax.ShapeDtypeStruct((K, DP), jnp.float32),
        ),
    )(batch_vectors, weight)

    result_pad, bh_flat = _sc_call(
        am,
        batch_keys_id.astype(jnp.int32),
        wt,
    )

    weight_new, cs_new, ea_new, hist_new, result = pl.pallas_call(
        _tc2_body,
        out_shape=(
            jax.ShapeDtypeStruct((D, K), jnp.float32),
            jax.ShapeDtypeStruct((K,), jnp.float32),
            jax.ShapeDtypeStruct((D, K), jnp.float32),
            jax.ShapeDtypeStruct((K, C), jnp.float32),
            jax.ShapeDtypeStruct((B, D), jnp.float32),
        ),
    )(batch_vectors, am, cluster_size, embed_avg, hist,
      bh_flat.reshape(K, CP), result_pad)

    return (result, am, weight_new, cs_new, ea_new, hist_new)


# speedup vs baseline: 1.0359x; 1.0359x over previous
import jax
import jax.numpy as jnp
from jax import lax
from jax.experimental import pallas as pl
from jax.experimental.pallas import tpu as pltpu
from jax.experimental.pallas import tpu_sc as plsc

B, D, K, C = 2048, 64, 512, 100
GAMMA = 0.99
EPS = 1e-05

NS = 16
L = 16
BPW = B // NS
CP = 128
HP = K * CP
HSL = HP // NS
DP = 128


def _tc1_body(x_ref, w_ref, am_ref, wtp_ref):
    x = x_ref[...]
    w = w_ref[...]

    xw = lax.dot_general(x, w, (((1,), (0,)), ((), ())),
                         precision=lax.Precision.HIGHEST)
    x2 = jnp.sum(x * x, axis=1, keepdims=True)
    w2 = jnp.sum(w * w, axis=0, keepdims=True)
    d2 = jnp.maximum(x2 - 2.0 * xw + w2, 0.0)
    dist = jnp.sqrt(d2)
    am = jnp.argmin(dist, axis=1)
    am_ref[...] = am

    wtp_ref[:, :D] = lax.transpose(w, (1, 0))


def _tc2_body(x_ref, am_ref, cs_ref, ea_ref,
              wn_ref, csn_ref, ean_ref):
    x = x_ref[...]
    am_col = am_ref[...][:, None]
    onehot = (am_col == lax.broadcasted_iota(jnp.int32, (1, K), 1)
              ).astype(jnp.float32)

    n_idx = jnp.sum(onehot, axis=0, keepdims=True)
    n_idx = jnp.where(n_idx == 0.0, 1.0, n_idx)
    cs_new = cs_ref[...][None, :] * GAMMA + (1.0 - GAMMA) * n_idx
    csn_ref[...] = cs_new[0]

    embed_sum = lax.dot_general(
        x, onehot, (((0,), (0,)), ((), ())),
        precision=lax.Precision.HIGHEST)
    ea_new = ea_ref[...] * GAMMA + (1.0 - GAMMA) * embed_sum
    ean_ref[...] = ea_new

    n = jnp.sum(cs_new)
    cs_smoothed = (cs_new + EPS) / (n + K * EPS) * n
    wn_ref[...] = ea_new / cs_smoothed


def _tc3_body(hist_ref, bh_ref, resp_ref, hn_ref, res_ref):
    hn_ref[...] = hist_ref[...] * GAMMA + (1.0 - GAMMA) * bh_ref[:, :C]
    res_ref[...] = resp_ref[:, :D]


def _sc_body(am_hbm, keys_hbm, wt_hbm,
             resp_hbm, bh_hbm,
             idx_v, flat_v, ones_v, rows_v, a_v, acc_sh, sem):
    cid = lax.axis_index("c")
    sid = lax.axis_index("s")
    base = sid * BPW

    @pl.when(cid == 1)
    def _gather():
        pltpu.sync_copy(am_hbm.at[pl.ds(base, BPW)], idx_v)
        pltpu.async_copy(wt_hbm.at[idx_v], rows_v, sem).wait()
        pltpu.sync_copy(rows_v, resp_hbm.at[pl.ds(base, BPW)])

    @pl.when(cid == 0)
    def _hist():
        pltpu.sync_copy(am_hbm.at[pl.ds(base, BPW)], idx_v)
        pltpu.sync_copy(keys_hbm.at[pl.ds(base, BPW)], flat_v)
        for j in range(BPW // L):
            s = pl.ds(j * L, L)
            flat_v[s] = idx_v[s] * CP + flat_v[s]
            ones_v[s] = jnp.full((L,), 1.0, jnp.float32)

        def _zero(i, _):
            a_v[pl.ds(i * L, L)] = jnp.zeros((L,), jnp.float32)
            return 0
        lax.fori_loop(0, HSL // L, _zero, 0)
        esl = pl.ds(sid * HSL, HSL)
        pltpu.sync_copy(a_v, acc_sh.at[esl])
        plsc.subcore_barrier()
        pltpu.sync_copy(ones_v, acc_sh.at[flat_v], add=True)
        plsc.subcore_barrier()
        pltpu.sync_copy(acc_sh.at[esl], a_v)
        pltpu.sync_copy(a_v, bh_hbm.at[esl])


_sc_call = pl.kernel(
    _sc_body,
    out_type=(jax.ShapeDtypeStruct((B, DP), jnp.float32),
              jax.ShapeDtypeStruct((HP,), jnp.float32)),
    mesh=plsc.VectorSubcoreMesh(core_axis_name="c", subcore_axis_name="s"),
    scratch_types=[
        pltpu.VMEM((BPW,), jnp.int32),
        pltpu.VMEM((BPW,), jnp.int32),
        pltpu.VMEM((BPW,), jnp.float32),
        pltpu.VMEM((BPW, DP), jnp.float32),
        pltpu.VMEM((HSL,), jnp.float32),
        pltpu.VMEM_SHARED((HP,), jnp.float32),
        pltpu.SemaphoreType.DMA,
    ],
)


def kernel(batch_vectors, batch_keys_id, weight, cluster_size, embed_avg, hist):
    am, wt = pl.pallas_call(
        _tc1_body,
        out_shape=(
            jax.ShapeDtypeStruct((B,), jnp.int32),
            jax.ShapeDtypeStruct((K, DP), jnp.float32),
        ),
    )(batch_vectors, weight)

    result_pad, bh_flat = _sc_call(
        am,
        batch_keys_id.astype(jnp.int32),
        wt,
    )

    weight_new, cs_new, ea_new = pl.pallas_call(
        _tc2_body,
        out_shape=(
            jax.ShapeDtypeStruct((D, K), jnp.float32),
            jax.ShapeDtypeStruct((K,), jnp.float32),
            jax.ShapeDtypeStruct((D, K), jnp.float32),
        ),
    )(batch_vectors, am, cluster_size, embed_avg)

    hist_new, result = pl.pallas_call(
        _tc3_body,
        out_shape=(
            jax.ShapeDtypeStruct((K, C), jnp.float32),
            jax.ShapeDtypeStruct((B, D), jnp.float32),
        ),
    )(hist, bh_flat.reshape(K, CP), result_pad)

    return (result, am, weight_new, cs_new, ea_new, hist_new)


# speedup vs baseline: 1.0873x; 1.0496x over previous
import jax
import jax.numpy as jnp
from jax import lax
from jax.experimental import pallas as pl
from jax.experimental.pallas import tpu as pltpu
from jax.experimental.pallas import tpu_sc as plsc

B, D, K, C = 2048, 64, 512, 100
GAMMA = 0.99
EPS = 1e-05

NS = 16
L = 16
BPW = B // NS
CP = 128
HP = K * CP
HSL = HP // NS
DP = 128


def _tc1_body(x_ref, w_ref, am_ref, wtp_ref):
    x = x_ref[...]
    w = w_ref[...]

    xw = lax.dot_general(x, w, (((1,), (0,)), ((), ())),
                         precision=lax.Precision.HIGHEST)
    x2 = jnp.sum(x * x, axis=1, keepdims=True)
    w2 = jnp.sum(w * w, axis=0, keepdims=True)
    d2 = jnp.maximum(x2 - 2.0 * xw + w2, 0.0)
    dist = jnp.sqrt(d2)
    am = jnp.argmin(dist, axis=1)
    am_ref[...] = am

    wtp_ref[:, :D] = lax.transpose(w, (1, 0))


def _tc2_body(x_ref, am_ref, cs_ref, ea_ref,
              wn_ref, csn_ref, ean_ref):
    x = x_ref[...]
    am_col = am_ref[...][:, None]
    onehot = (am_col == lax.broadcasted_iota(jnp.int32, (1, K), 1)
              ).astype(jnp.float32)

    n_idx = jnp.sum(onehot, axis=0, keepdims=True)
    n_idx = jnp.where(n_idx == 0.0, 1.0, n_idx)
    cs_new = cs_ref[...][None, :] * GAMMA + (1.0 - GAMMA) * n_idx
    csn_ref[...] = cs_new[0]

    embed_sum = lax.dot_general(
        x, onehot, (((0,), (0,)), ((), ())))
    ea_new = ea_ref[...] * GAMMA + (1.0 - GAMMA) * embed_sum
    ean_ref[...] = ea_new

    n = jnp.sum(cs_new)
    cs_smoothed = (cs_new + EPS) / (n + K * EPS) * n
    wn_ref[...] = ea_new / cs_smoothed


def _tc3_body(hist_ref, bh_ref, resp_ref, hn_ref, res_ref):
    bh = bh_ref[...].reshape(K, CP)
    hn_ref[...] = hist_ref[...] * GAMMA + (1.0 - GAMMA) * bh[:, :C]
    res_ref[...] = resp_ref[:, :D]


def _sc_body(am_hbm, keys_hbm, wt_hbm,
             resp_hbm, bh_hbm,
             idx_v, flat_v, ones_v, rows_v, a_v, acc_sh, sem):
    cid = lax.axis_index("c")
    sid = lax.axis_index("s")
    base = sid * BPW

    @pl.when(cid == 1)
    def _gather():
        pltpu.sync_copy(am_hbm.at[pl.ds(base, BPW)], idx_v)
        pltpu.async_copy(wt_hbm.at[idx_v], rows_v, sem).wait()
        pltpu.sync_copy(rows_v, resp_hbm.at[pl.ds(base, BPW)])

    @pl.when(cid == 0)
    def _hist():
        pltpu.sync_copy(am_hbm.at[pl.ds(base, BPW)], idx_v)
        pltpu.sync_copy(keys_hbm.at[pl.ds(base, BPW)], flat_v)
        for j in range(BPW // L):
            s = pl.ds(j * L, L)
            flat_v[s] = idx_v[s] * CP + flat_v[s]
            ones_v[s] = jnp.full((L,), 1.0, jnp.float32)

        def _zero(i, _):
            a_v[pl.ds(i * L, L)] = jnp.zeros((L,), jnp.float32)
            return 0
        lax.fori_loop(0, HSL // L, _zero, 0)
        esl = pl.ds(sid * HSL, HSL)
        pltpu.sync_copy(a_v, acc_sh.at[esl])
        plsc.subcore_barrier()
        pltpu.sync_copy(ones_v, acc_sh.at[flat_v], add=True)
        plsc.subcore_barrier()
        pltpu.sync_copy(acc_sh.at[esl], a_v)
        pltpu.sync_copy(a_v, bh_hbm.at[esl])


_sc_call = pl.kernel(
    _sc_body,
    out_type=(jax.ShapeDtypeStruct((B, DP), jnp.float32),
              jax.ShapeDtypeStruct((HP,), jnp.float32)),
    mesh=plsc.VectorSubcoreMesh(core_axis_name="c", subcore_axis_name="s"),
    scratch_types=[
        pltpu.VMEM((BPW,), jnp.int32),
        pltpu.VMEM((BPW,), jnp.int32),
        pltpu.VMEM((BPW,), jnp.float32),
        pltpu.VMEM((BPW, DP), jnp.float32),
        pltpu.VMEM((HSL,), jnp.float32),
        pltpu.VMEM_SHARED((HP,), jnp.float32),
        pltpu.SemaphoreType.DMA,
    ],
)


def kernel(batch_vectors, batch_keys_id, weight, cluster_size, embed_avg, hist):
    am, wt = pl.pallas_call(
        _tc1_body,
        out_shape=(
            jax.ShapeDtypeStruct((B,), jnp.int32),
            jax.ShapeDtypeStruct((K, DP), jnp.float32),
        ),
    )(batch_vectors, weight)

    result_pad, bh_flat = _sc_call(
        am,
        batch_keys_id.astype(jnp.int32),
        wt,
    )

    weight_new, cs_new, ea_new = pl.pallas_call(
        _tc2_body,
        out_shape=(
            jax.ShapeDtypeStruct((D, K), jnp.float32),
            jax.ShapeDtypeStruct((K,), jnp.float32),
            jax.ShapeDtypeStruct((D, K), jnp.float32),
        ),
    )(batch_vectors, am, cluster_size, embed_avg)

    hist_new, result = pl.pallas_call(
        _tc3_body,
        out_shape=(
            jax.ShapeDtypeStruct((K, C), jnp.float32),
            jax.ShapeDtypeStruct((B, D), jnp.float32),
        ),
    )(hist, bh_flat, result_pad)

    return (result, am, weight_new, cs_new, ea_new, hist_new)
